# baseline (speedup 1.0000x reference)
import functools

import jax
import jax.numpy as jnp
from jax.experimental import pallas as pl
from jax.experimental.pallas import tpu as pltpu

B = 256
N = 202
FEAT = 6
NH = 128
K_HOP = 2
NUM_CAT = 3
ALPHA = 0.5

BB = 8
NBUF = 3
NP = 208
M = BB * NP


def _lrelu(x):
    return jnp.where(x >= 0, x, 0.2 * x)


def _encoder_kernel(nf_ref, adj_hbm, wemb_ref, wcat_ref, out_ref,
                    abuf, abf, hbuf, ybuf, msgbuf, sem):
    s = pl.program_id(0)
    nsteps = pl.num_programs(0)

    H = BB // 2

    def copies(step, slot_):
        return [pltpu.make_async_copy(
            adj_hbm.at[pl.ds(step * BB + k * H, H)],
            abuf.at[slot_, pl.ds(k * H, H)],
            sem.at[slot_, k]) for k in range(2)]

    def start(step, slot_):
        for c in copies(step, slot_):
            c.start()

    @pl.when(s == 0)
    def _():
        start(0, 0)
        start(1, 1)

    @pl.when(s + 2 < nsteps)
    def _():
        start(s + 2, jax.lax.rem(s + 2, NBUF))

    slot = jax.lax.rem(s, NBUF)
    for c in copies(s, slot):
        c.wait()

    inv_degs = []
    for b in range(BB):
        adj = abuf[slot, b]
        inv_degs.append(1.0 / (jnp.sum(adj, axis=-1, keepdims=True) + 1e-6))
        abf[b] = adj.astype(jnp.bfloat16)
        hbuf[b * NP:b * NP + N, :] = jnp.dot(
            nf_ref[b], wemb_ref[...], preferred_element_type=jnp.float32)

    for hop in range(K_HOP):
        ybuf[...] = jnp.dot(hbuf[...].astype(jnp.bfloat16), wcat_ref[hop],
                            preferred_element_type=jnp.float32
                            ).astype(jnp.bfloat16)
        for b in range(BB):
            msg = None
            for c in range(NUM_CAT):
                m = jnp.dot(abf[b, c],
                            ybuf[b * NP:b * NP + N, c * NH:(c + 1) * NH],
                            preferred_element_type=jnp.float32) * inv_degs[b][c]
                msg = m if msg is None else msg + m
            msgbuf[b * NP:b * NP + N, :] = msg
        pre = (ybuf[:, NUM_CAT * NH:].astype(jnp.float32)
               + msgbuf[...] * (1.0 / NUM_CAT))
        hbuf[...] = ALPHA * hbuf[...] + _lrelu(pre)

    for b in range(BB):
        out_ref[b] = jnp.mean(hbuf[b * NP:b * NP + N, :], axis=0,
                              keepdims=True)


def _head_kernel(hm_ref,
                 wp1, bp1, wp2, bp2, wp3, bp3,
                 wm1, bm1, wm2, bm2, wm3, bm3,
                 ws1, bs1, ws2, bs2, ws3, bs3,
                 out_ref):
    x = hm_ref[...]
    x = _lrelu(jnp.dot(x, wp1[...], preferred_element_type=jnp.float32) + bp1[...])
    x = _lrelu(jnp.dot(x, wp2[...], preferred_element_type=jnp.float32) + bp2[...])
    x = jnp.dot(x, wp3[...], preferred_element_type=jnp.float32) + bp3[...]
    mean = x[:, :NH]
    std = x[:, NH:]
    m = _lrelu(jnp.dot(mean, wm1[...], preferred_element_type=jnp.float32) + bm1[...])
    m = _lrelu(jnp.dot(m, wm2[...], preferred_element_type=jnp.float32) + bm2[...])
    m = jnp.dot(m, wm3[...], preferred_element_type=jnp.float32) + bm3[...]
    s = _lrelu(jnp.dot(std, ws1[...], preferred_element_type=jnp.float32) + bs1[...])
    s = _lrelu(jnp.dot(s, ws2[...], preferred_element_type=jnp.float32) + bs2[...])
    s = jnp.dot(s, ws3[...], preferred_element_type=jnp.float32) + bs3[...]
    s = jnp.maximum(s, 0.0) + jnp.log1p(jnp.exp(-jnp.abs(s))) + 1e-5
    out_ref[:, :NH] = m
    out_ref[:, NH:] = s


@functools.partial(jax.jit, static_argnames=())
def kernel(node_features, het_adj, W_emb, W_gcn, W_self,
           Wp1, bp1, Wp2, bp2, Wp3, bp3,
           Wm1, bm1, Wm2, bm2, Wm3, bm3,
           Ws1, bs1, Ws2, bs2, Ws3, bs3):
    W_cat = jnp.concatenate([W_gcn, W_self[:, None]], axis=1)
    W_cat = jnp.moveaxis(W_cat, 1, 2).reshape(K_HOP, NH, 4 * NH)
    W_cat = W_cat.astype(jnp.bfloat16)

    h_mean = pl.pallas_call(
        _encoder_kernel,
        grid=(B // BB,),
        in_specs=[
            pl.BlockSpec((BB, N, FEAT), lambda b: (b, 0, 0)),
            pl.BlockSpec(memory_space=pltpu.MemorySpace.HBM),
            pl.BlockSpec((FEAT, NH), lambda b: (0, 0)),
            pl.BlockSpec((K_HOP, NH, 4 * NH), lambda b: (0, 0, 0)),
        ],
        out_specs=pl.BlockSpec((BB, 1, NH), lambda b: (b, 0, 0)),
        out_shape=jax.ShapeDtypeStruct((B, 1, NH), jnp.float32),
        scratch_shapes=[
            pltpu.VMEM((NBUF, BB, NUM_CAT, N, N), jnp.float32),
            pltpu.VMEM((BB, NUM_CAT, N, N), jnp.bfloat16),
            pltpu.VMEM((M, NH), jnp.float32),
            pltpu.VMEM((M, 4 * NH), jnp.bfloat16),
            pltpu.VMEM((M, NH), jnp.float32),
            pltpu.SemaphoreType.DMA((NBUF, 2)),
        ],
        compiler_params=pltpu.CompilerParams(
            dimension_semantics=("arbitrary",),
        ),
    )(node_features, het_adj, W_emb, W_cat)
    h_mean = h_mean.reshape(B, NH)

    biases = [b.reshape(1, -1) for b in
              (bp1, bp2, bp3, bm1, bm2, bm3, bs1, bs2, bs3)]
    bp1r, bp2r, bp3r, bm1r, bm2r, bm3r, bs1r, bs2r, bs3r = biases

    out = pl.pallas_call(
        _head_kernel,
        out_shape=jax.ShapeDtypeStruct((B, 2 * NH), jnp.float32),
    )(h_mean,
      Wp1, bp1r, Wp2, bp2r, Wp3, bp3r,
      Wm1, bm1r, Wm2, bm2r, Wm3, bm3r,
      Ws1, bs1r, Ws2, bs2r, Ws3, bs3r)
    return out


# speedup vs baseline: 1.0099x; 1.0099x over previous
import functools

import jax
import jax.numpy as jnp
from jax.experimental import pallas as pl
from jax.experimental.pallas import tpu as pltpu

B = 256
N = 202
FEAT = 6
NH = 128
K_HOP = 2
NUM_CAT = 3
ALPHA = 0.5

BB = 8
NBUF = 3
NP = 208
M = BB * NP


def _lrelu(x):
    return jnp.where(x >= 0, x, 0.2 * x)


def _encoder_kernel(nf_ref, adj_hbm, wemb_ref, wcat_ref, out_ref,
                    abuf, abf, hbuf, ybuf, sem):
    s = pl.program_id(0)
    nsteps = pl.num_programs(0)

    def copy(step, slot_):
        return pltpu.make_async_copy(
            adj_hbm.at[pl.ds(step * BB, BB)], abuf.at[slot_], sem.at[slot_])

    @pl.when(s == 0)
    def _():
        copy(0, 0).start()
        copy(1, 1).start()

    @pl.when(s + 2 < nsteps)
    def _():
        copy(s + 2, jax.lax.rem(s + 2, NBUF)).start()

    slot = jax.lax.rem(s, NBUF)
    copy(s, slot).wait()

    inv_degs = []
    for b in range(BB):
        adj = abuf[slot, b]
        inv_degs.append(1.0 / (jnp.sum(adj, axis=-1, keepdims=True) + 1e-6))
        abf[b] = adj.astype(jnp.bfloat16)
        hbuf[b * NP:b * NP + N, :] = jnp.dot(
            nf_ref[b], wemb_ref[...], preferred_element_type=jnp.float32)

    for hop in range(K_HOP):
        ybuf[...] = jnp.dot(hbuf[...].astype(jnp.bfloat16), wcat_ref[hop],
                            preferred_element_type=jnp.float32
                            ).astype(jnp.bfloat16)
        for b in range(BB):
            msg = None
            for c in range(NUM_CAT):
                m = jnp.dot(abf[b, c],
                            ybuf[b * NP:b * NP + N, c * NH:(c + 1) * NH],
                            preferred_element_type=jnp.float32) * inv_degs[b][c]
                msg = m if msg is None else msg + m
            rows = pl.ds(b * NP, N)
            pre = (ybuf[rows, NUM_CAT * NH:].astype(jnp.float32)
                   + msg * (1.0 / NUM_CAT))
            hbuf[rows, :] = ALPHA * hbuf[rows, :] + _lrelu(pre)

    for b in range(BB):
        out_ref[b] = jnp.mean(hbuf[b * NP:b * NP + N, :], axis=0,
                              keepdims=True)


def _head_kernel(hm_ref,
                 wp1, bp1, wp2, bp2, wp3, bp3,
                 wm1, bm1, wm2, bm2, wm3, bm3,
                 ws1, bs1, ws2, bs2, ws3, bs3,
                 out_ref):
    x = hm_ref[...]
    x = _lrelu(jnp.dot(x, wp1[...], preferred_element_type=jnp.float32) + bp1[...])
    x = _lrelu(jnp.dot(x, wp2[...], preferred_element_type=jnp.float32) + bp2[...])
    x = jnp.dot(x, wp3[...], preferred_element_type=jnp.float32) + bp3[...]
    mean = x[:, :NH]
    std = x[:, NH:]
    m = _lrelu(jnp.dot(mean, wm1[...], preferred_element_type=jnp.float32) + bm1[...])
    m = _lrelu(jnp.dot(m, wm2[...], preferred_element_type=jnp.float32) + bm2[...])
    m = jnp.dot(m, wm3[...], preferred_element_type=jnp.float32) + bm3[...]
    s = _lrelu(jnp.dot(std, ws1[...], preferred_element_type=jnp.float32) + bs1[...])
    s = _lrelu(jnp.dot(s, ws2[...], preferred_element_type=jnp.float32) + bs2[...])
    s = jnp.dot(s, ws3[...], preferred_element_type=jnp.float32) + bs3[...]
    s = jnp.maximum(s, 0.0) + jnp.log1p(jnp.exp(-jnp.abs(s))) + 1e-5
    out_ref[:, :NH] = m
    out_ref[:, NH:] = s


@functools.partial(jax.jit, static_argnames=())
def kernel(node_features, het_adj, W_emb, W_gcn, W_self,
           Wp1, bp1, Wp2, bp2, Wp3, bp3,
           Wm1, bm1, Wm2, bm2, Wm3, bm3,
           Ws1, bs1, Ws2, bs2, Ws3, bs3):
    W_cat = jnp.concatenate([W_gcn, W_self[:, None]], axis=1)
    W_cat = jnp.moveaxis(W_cat, 1, 2).reshape(K_HOP, NH, 4 * NH)
    W_cat = W_cat.astype(jnp.bfloat16)

    h_mean = pl.pallas_call(
        _encoder_kernel,
        grid=(B // BB,),
        in_specs=[
            pl.BlockSpec((BB, N, FEAT), lambda b: (b, 0, 0)),
            pl.BlockSpec(memory_space=pltpu.MemorySpace.HBM),
            pl.BlockSpec((FEAT, NH), lambda b: (0, 0)),
            pl.BlockSpec((K_HOP, NH, 4 * NH), lambda b: (0, 0, 0)),
        ],
        out_specs=pl.BlockSpec((BB, 1, NH), lambda b: (b, 0, 0)),
        out_shape=jax.ShapeDtypeStruct((B, 1, NH), jnp.float32),
        scratch_shapes=[
            pltpu.VMEM((NBUF, BB, NUM_CAT, N, N), jnp.float32),
            pltpu.VMEM((BB, NUM_CAT, N, N), jnp.bfloat16),
            pltpu.VMEM((M, NH), jnp.float32),
            pltpu.VMEM((M, 4 * NH), jnp.bfloat16),
            pltpu.SemaphoreType.DMA((NBUF,)),
        ],
        compiler_params=pltpu.CompilerParams(
            dimension_semantics=("arbitrary",),
        ),
    )(node_features, het_adj, W_emb, W_cat)
    h_mean = h_mean.reshape(B, NH)

    biases = [b.reshape(1, -1) for b in
              (bp1, bp2, bp3, bm1, bm2, bm3, bs1, bs2, bs3)]
    bp1r, bp2r, bp3r, bm1r, bm2r, bm3r, bs1r, bs2r, bs3r = biases

    out = pl.pallas_call(
        _head_kernel,
        out_shape=jax.ShapeDtypeStruct((B, 2 * NH), jnp.float32),
    )(h_mean,
      Wp1, bp1r, Wp2, bp2r, Wp3, bp3r,
      Wm1, bm1r, Wm2, bm2r, Wm3, bm3r,
      Ws1, bs1r, Ws2, bs2r, Ws3, bs3r)
    return out


# speedup vs baseline: 1.0154x; 1.0055x over previous
import functools

import jax
import jax.numpy as jnp
from jax.experimental import pallas as pl
from jax.experimental.pallas import tpu as pltpu

B = 256
N = 202
FEAT = 6
NH = 128
K_HOP = 2
NUM_CAT = 3
ALPHA = 0.5

BB = 16
NBUF = 3
NP = 208
M = BB * NP


def _lrelu(x):
    return jnp.where(x >= 0, x, 0.2 * x)


def _encoder_kernel(nf_ref, adj_hbm, wemb_ref, wcat_ref, out_ref,
                    abuf, abf, hbuf, ybuf, sem):
    s = pl.program_id(0)
    nsteps = pl.num_programs(0)

    def copy(step, slot_):
        return pltpu.make_async_copy(
            adj_hbm.at[pl.ds(step * BB, BB)], abuf.at[slot_], sem.at[slot_])

    @pl.when(s == 0)
    def _():
        copy(0, 0).start()
        copy(1, 1).start()

    @pl.when(s + 2 < nsteps)
    def _():
        copy(s + 2, jax.lax.rem(s + 2, NBUF)).start()

    slot = jax.lax.rem(s, NBUF)
    copy(s, slot).wait()

    inv_degs = []
    for b in range(BB):
        adj = abuf[slot, b]
        inv_degs.append(1.0 / (jnp.sum(adj, axis=-1, keepdims=True) + 1e-6))
        abf[b] = adj.astype(jnp.bfloat16)
        hbuf[b * NP:b * NP + N, :] = jnp.dot(
            nf_ref[b], wemb_ref[...], preferred_element_type=jnp.float32)

    for hop in range(K_HOP):
        ybuf[...] = jnp.dot(hbuf[...].astype(jnp.bfloat16), wcat_ref[hop],
                            preferred_element_type=jnp.float32
                            ).astype(jnp.bfloat16)
        for b in range(BB):
            msg = None
            for c in range(NUM_CAT):
                m = jnp.dot(abf[b, c],
                            ybuf[b * NP:b * NP + N, c * NH:(c + 1) * NH],
                            preferred_element_type=jnp.float32) * inv_degs[b][c]
                msg = m if msg is None else msg + m
            rows = pl.ds(b * NP, N)
            pre = (ybuf[rows, NUM_CAT * NH:].astype(jnp.float32)
                   + msg * (1.0 / NUM_CAT))
            hbuf[rows, :] = ALPHA * hbuf[rows, :] + _lrelu(pre)

    for b in range(BB):
        out_ref[b] = jnp.mean(hbuf[b * NP:b * NP + N, :], axis=0,
                              keepdims=True)


def _head_kernel(hm_ref,
                 wp1, bp1, wp2, bp2, wp3, bp3,
                 wm1, bm1, wm2, bm2, wm3, bm3,
                 ws1, bs1, ws2, bs2, ws3, bs3,
                 out_ref):
    x = hm_ref[...]
    x = _lrelu(jnp.dot(x, wp1[...], preferred_element_type=jnp.float32) + bp1[...])
    x = _lrelu(jnp.dot(x, wp2[...], preferred_element_type=jnp.float32) + bp2[...])
    x = jnp.dot(x, wp3[...], preferred_element_type=jnp.float32) + bp3[...]
    mean = x[:, :NH]
    std = x[:, NH:]
    m = _lrelu(jnp.dot(mean, wm1[...], preferred_element_type=jnp.float32) + bm1[...])
    m = _lrelu(jnp.dot(m, wm2[...], preferred_element_type=jnp.float32) + bm2[...])
    m = jnp.dot(m, wm3[...], preferred_element_type=jnp.float32) + bm3[...]
    s = _lrelu(jnp.dot(std, ws1[...], preferred_element_type=jnp.float32) + bs1[...])
    s = _lrelu(jnp.dot(s, ws2[...], preferred_element_type=jnp.float32) + bs2[...])
    s = jnp.dot(s, ws3[...], preferred_element_type=jnp.float32) + bs3[...]
    s = jnp.maximum(s, 0.0) + jnp.log1p(jnp.exp(-jnp.abs(s))) + 1e-5
    out_ref[:, :NH] = m
    out_ref[:, NH:] = s


@functools.partial(jax.jit, static_argnames=())
def kernel(node_features, het_adj, W_emb, W_gcn, W_self,
           Wp1, bp1, Wp2, bp2, Wp3, bp3,
           Wm1, bm1, Wm2, bm2, Wm3, bm3,
           Ws1, bs1, Ws2, bs2, Ws3, bs3):
    W_cat = jnp.concatenate([W_gcn, W_self[:, None]], axis=1)
    W_cat = jnp.moveaxis(W_cat, 1, 2).reshape(K_HOP, NH, 4 * NH)
    W_cat = W_cat.astype(jnp.bfloat16)

    h_mean = pl.pallas_call(
        _encoder_kernel,
        grid=(B // BB,),
        in_specs=[
            pl.BlockSpec((BB, N, FEAT), lambda b: (b, 0, 0)),
            pl.BlockSpec(memory_space=pltpu.MemorySpace.HBM),
            pl.BlockSpec((FEAT, NH), lambda b: (0, 0)),
            pl.BlockSpec((K_HOP, NH, 4 * NH), lambda b: (0, 0, 0)),
        ],
        out_specs=pl.BlockSpec((BB, 1, NH), lambda b: (b, 0, 0)),
        out_shape=jax.ShapeDtypeStruct((B, 1, NH), jnp.float32),
        scratch_shapes=[
            pltpu.VMEM((NBUF, BB, NUM_CAT, N, N), jnp.float32),
            pltpu.VMEM((BB, NUM_CAT, N, N), jnp.bfloat16),
            pltpu.VMEM((M, NH), jnp.float32),
            pltpu.VMEM((M, 4 * NH), jnp.bfloat16),
            pltpu.SemaphoreType.DMA((NBUF,)),
        ],
        compiler_params=pltpu.CompilerParams(
            dimension_semantics=("arbitrary",),
        ),
    )(node_features, het_adj, W_emb, W_cat)
    h_mean = h_mean.reshape(B, NH)

    biases = [b.reshape(1, -1) for b in
              (bp1, bp2, bp3, bm1, bm2, bm3, bs1, bs2, bs3)]
    bp1r, bp2r, bp3r, bm1r, bm2r, bm3r, bs1r, bs2r, bs3r = biases

    out = pl.pallas_call(
        _head_kernel,
        out_shape=jax.ShapeDtypeStruct((B, 2 * NH), jnp.float32),
    )(h_mean,
      Wp1, bp1r, Wp2, bp2r, Wp3, bp3r,
      Wm1, bm1r, Wm2, bm2r, Wm3, bm3r,
      Ws1, bs1r, Ws2, bs2r, Ws3, bs3r)
    return out
